# baseline (speedup 1.0000x reference)
import functools

import jax
import jax.numpy as jnp
from jax import lax
from jax.experimental import pallas as pl
from jax.experimental.pallas import tpu as pltpu
from jax.experimental.pallas import tpu_sc as plsc

NC = 2
NS = 16
L = 16
CH = 64


def _prep_body(n, n_pad, fs_ref, fd_ref, al_ref, ar_ref,
               el_ref, er_ref, g_ref):
    el = jnp.sum(fs_ref[...] * al_ref[...], axis=1)
    er = jnp.sum(fd_ref[...] * ar_ref[...], axis=1)
    el_ref[pl.ds(0, n)] = el
    er_ref[pl.ds(0, n)] = er
    zpad = jnp.zeros((n_pad - n,), jnp.float32)
    el_ref[pl.ds(n, n_pad - n)] = zpad
    er_ref[pl.ds(n, n_pad - n)] = zpad
    b = jnp.max(el) + jnp.max(er)
    g = jnp.where(b > 0, b, 0.01 * b)
    g_ref[...] = jnp.full((L,), g, jnp.float32)


def _finalize_body(n, acc_ref, s_ref, o_ref):
    a = acc_ref[0] + acc_ref[1]
    ss = s_ref[0] + s_ref[1]
    denom = jnp.where(ss == 0.0, 1.0, ss)
    o_ref[...] = a[:n] / denom[:n, None]


def _make_sc_kernel(n, e_pad, d, n_pad):
    epw = e_pad // (NC * NS)
    nch = epw // CH
    rpt = n_pad // NS
    assert nch % 3 == 1 and nch > 3
    assert rpt % CH == 0
    mesh = plsc.VectorSubcoreMesh(
        core_axis_name="c", subcore_axis_name="s",
        num_cores=NC, num_subcores=NS)

    buf_types = []
    for _ in range(3):
        buf_types += [
            pltpu.VMEM((CH,), jnp.int32),
            pltpu.VMEM((CH,), jnp.int32),
            pltpu.VMEM((CH,), jnp.float32),
            pltpu.VMEM((CH,), jnp.float32),
            pltpu.VMEM((CH, d), jnp.float32),
            pltpu.SemaphoreType.DMA,
            pltpu.SemaphoreType.DMA,
        ]

    @functools.partial(
        pl.kernel,
        out_type=[
            jax.ShapeDtypeStruct((NC * n_pad, d), jnp.float32),
            jax.ShapeDtypeStruct((NC * n_pad,), jnp.float32),
        ],
        mesh=mesh,
        compiler_params=pltpu.CompilerParams(needs_layout_passes=False),
        scratch_types=[
            pltpu.VMEM((epw // 2,), jnp.int32),
            pltpu.VMEM((epw // 2,), jnp.int32),
        ] + buf_types + [
            pltpu.VMEM((CH,), jnp.float32),
            pltpu.VMEM((L,), jnp.float32),
            pltpu.VMEM((rpt,), jnp.float32),
            pltpu.VMEM_SHARED((n_pad, d), jnp.float32),
            pltpu.VMEM_SHARED((n_pad,), jnp.float32),
        ],
    )
    def sc_kernel(src_hbm, dst_hbm, el_hbm, er_hbm, g_hbm, feat_hbm,
                  z2_hbm, z1_hbm, acc_out, s_out, src_v, dst_v, *refs):
        bufs = [refs[7 * i:7 * i + 7] for i in range(3)]
        p_v, g_v, st1_v, acc_sp, s_sp = refs[21:]
        st2_v = bufs[0][4]
        cid = lax.axis_index("c")
        tid = lax.axis_index("s")
        base = (cid * NS + tid) * epw

        pltpu.sync_copy(z2_hbm, st2_v)
        pltpu.sync_copy(z1_hbm, st1_v)
        for k in range(rpt // CH):
            pltpu.sync_copy(st2_v,
                            acc_sp.at[pl.ds(tid * rpt + k * CH, CH), :])
        pltpu.sync_copy(st1_v, s_sp.at[pl.ds(tid * rpt, rpt)])

        base2 = (cid * NS + tid) * (epw // 2)
        pltpu.sync_copy(src_hbm.at[pl.ds(base2, epw // 2)], src_v)
        pltpu.sync_copy(dst_hbm.at[pl.ds(base2, epw // 2)], dst_v)
        pltpu.sync_copy(g_hbm, g_v)
        gvec = g_v[...]

        def decode_fire(c, b):
            srcc, dstc, elg, erg, rows, sem_g, _ = bufs[b]
            for m in range(CH // 32):
                s32 = src_v[pl.ds(c * (CH // 2) + m * L, L)]
                d32 = dst_v[pl.ds(c * (CH // 2) + m * L, L)]
                srcc[pl.ds(m * 32, L)] = s32 & 0xFFFF
                srcc[pl.ds(m * 32 + L, L)] = lax.shift_right_logical(s32, 16)
                dstc[pl.ds(m * 32, L)] = d32 & 0xFFFF
                dstc[pl.ds(m * 32 + L, L)] = lax.shift_right_logical(d32, 16)
            pltpu.async_copy(el_hbm.at[srcc], elg, sem_g)
            pltpu.async_copy(er_hbm.at[dstc], erg, sem_g)
            pltpu.async_copy(feat_hbm.at[srcc], rows, sem_g)

        def wait_gather(b):
            srcc, dstc, elg, erg, rows, sem_g, _ = bufs[b]
            pltpu.make_async_copy(el_hbm.at[srcc], elg, sem_g).wait()
            pltpu.make_async_copy(er_hbm.at[dstc], erg, sem_g).wait()
            pltpu.make_async_copy(feat_hbm.at[srcc], rows, sem_g).wait()

        def wait_scatter(b):
            _, dstc, _, _, rows, _, sem_s = bufs[b]
            pltpu.make_async_copy(rows, acc_sp.at[dstc], sem_s).wait()

        def compute_scale(b):
            srcc, dstc, elg, erg, rows, _, _ = bufs[b]
            for v in range(CH // L):
                sl = pl.ds(v * L, L)
                e16 = elg[sl] + erg[sl]
                e16 = jnp.where(e16 > 0, e16, 0.01 * e16)
                p_v[sl] = jnp.exp(e16 - gvec)
            pltpu.sync_copy(p_v, s_sp.at[dstc], add=True)
            for vb in range(CH // L):
                p16 = p_v[pl.ds(vb * L, L)]
                for jj in range(L):
                    j = vb * L + jj
                    a = p16[jj]
                    for r in range(d // L):
                        sl = pl.ds(r * L, L)
                        rows[j, sl] = rows[j, sl] * a

        def fire_scatter(b):
            _, dstc, _, _, rows, _, sem_s = bufs[b]
            pltpu.async_copy(rows, acc_sp.at[dstc], sem_s, add=True)

        plsc.subcore_barrier()

        decode_fire(0, 0)

        def outer(c2, carry):
            for j in range(3):
                c = 3 * c2 + j
                nxt = (j + 1) % 3
                wait_gather(j)
                if j == 2:
                    wait_scatter(nxt)
                else:
                    @pl.when(c2 >= 1)
                    def _():
                        wait_scatter(nxt)
                decode_fire(c + 1, nxt)
                compute_scale(j)
                fire_scatter(j)
            return carry

        lax.fori_loop(0, (nch - 1) // 3, outer, 0)

        wait_gather(0)
        compute_scale(0)
        wait_scatter(1)
        wait_scatter(2)
        pltpu.sync_copy(bufs[0][4], acc_sp.at[bufs[0][1]], add=True)

        plsc.subcore_barrier()

        o = cid * n_pad + tid * rpt
        pltpu.sync_copy(s_sp.at[pl.ds(tid * rpt, rpt)], st1_v)
        pltpu.sync_copy(st1_v, s_out.at[pl.ds(o, rpt)])
        for k in range(rpt // CH):
            pltpu.sync_copy(acc_sp.at[pl.ds(tid * rpt + k * CH, CH), :],
                            st2_v)
            pltpu.sync_copy(st2_v, acc_out.at[pl.ds(o + k * CH, CH), :])

    return sc_kernel


def kernel(edge_index, feat_src, feat_dst, attn_l, attn_r):
    n, d = feat_src.shape
    e = edge_index.shape[1]
    n_pad = -(-n // (NS * CH)) * (NS * CH)
    nw = NC * NS
    epw = -(-e // (nw * 256)) * 256
    while (epw // CH) % 3 != 1:
        epw += 256
    e_pad = nw * epw
    assert n_pad < (1 << 15)

    pad = e_pad - e
    src16 = jnp.concatenate(
        [edge_index[0], jnp.zeros((pad,), jnp.int32)]).astype(jnp.int16)
    dst16 = jnp.concatenate(
        [edge_index[1], jnp.full((pad,), n, jnp.int32)]).astype(jnp.int16)
    src16 = lax.bitcast_convert_type(src16.reshape(-1, 2), jnp.int32)
    dst16 = lax.bitcast_convert_type(dst16.reshape(-1, 2), jnp.int32)

    el, er, g = pl.pallas_call(
        functools.partial(_prep_body, n, n_pad),
        out_shape=[
            jax.ShapeDtypeStruct((n_pad,), jnp.float32),
            jax.ShapeDtypeStruct((n_pad,), jnp.float32),
            jax.ShapeDtypeStruct((L,), jnp.float32),
        ],
    )(feat_src, feat_dst, attn_l, attn_r)

    rpt = n_pad // NS
    z2 = jnp.zeros((CH, d), jnp.float32)
    z1 = jnp.zeros((rpt,), jnp.float32)

    sc_kernel = _make_sc_kernel(n, e_pad, d, n_pad)
    acc2, s2 = sc_kernel(src16, dst16, el, er, g, feat_src, z2, z1)

    out = pl.pallas_call(
        functools.partial(_finalize_body, n),
        out_shape=jax.ShapeDtypeStruct((n, d), jnp.float32),
    )(acc2.reshape(NC, n_pad, d), s2.reshape(NC, n_pad))
    return out


# speedup vs baseline: 2.8967x; 2.8967x over previous
import functools

import jax
import jax.numpy as jnp
from jax import lax
from jax.experimental import pallas as pl
from jax.experimental.pallas import tpu as pltpu
from jax.experimental.pallas import tpu_sc as plsc

NC = 2
NS = 16
L = 16
CH = 80


def _prep_body(fs_ref, fd_ref, al_ref, ar_ref, el_ref, er_ref, g_ref):
    el = jnp.sum(fs_ref[...] * al_ref[...], axis=1)
    er = jnp.sum(fd_ref[...] * ar_ref[...], axis=1)
    el_ref[...] = el
    er_ref[...] = er
    b = jnp.max(el) + jnp.max(er)
    g = jnp.where(b > 0, b, 0.01 * b)
    g_ref[...] = jnp.full((L,), g, jnp.float32)


def _finalize_body(n, acc_ref, s_ref, o_ref):
    a = acc_ref[0] + acc_ref[1]
    ss = s_ref[0] + s_ref[1]
    denom = jnp.where(ss == 0.0, 1.0, ss)
    o_ref[...] = a[:n] / denom[:n, None]


def _make_sc_kernel(n, e, d, n_pad):
    epw = e // (NC * NS)
    nch = epw // CH
    rpt = n_pad // NS
    assert nch % 2 == 1
    mesh = plsc.VectorSubcoreMesh(
        core_axis_name="c", subcore_axis_name="s",
        num_cores=NC, num_subcores=NS)

    @functools.partial(
        pl.kernel,
        out_type=[
            jax.ShapeDtypeStruct((NC * n_pad, d), jnp.float32),
            jax.ShapeDtypeStruct((NC * n_pad,), jnp.float32),
        ],
        mesh=mesh,
        compiler_params=pltpu.CompilerParams(needs_layout_passes=False),
        scratch_types=[
            pltpu.VMEM((epw,), jnp.int32),
            pltpu.VMEM((epw,), jnp.int32),
            pltpu.VMEM((CH,), jnp.int32),
            pltpu.VMEM((CH,), jnp.int32),
            pltpu.VMEM((CH,), jnp.float32),
            pltpu.VMEM((CH,), jnp.float32),
            pltpu.VMEM((CH, d), jnp.float32),
            pltpu.SemaphoreType.DMA,
            pltpu.VMEM((CH,), jnp.int32),
            pltpu.VMEM((CH,), jnp.int32),
            pltpu.VMEM((CH,), jnp.float32),
            pltpu.VMEM((CH,), jnp.float32),
            pltpu.VMEM((CH, d), jnp.float32),
            pltpu.SemaphoreType.DMA,
            pltpu.VMEM((CH,), jnp.float32),
            pltpu.VMEM((L,), jnp.float32),
            pltpu.VMEM((rpt,), jnp.float32),
            pltpu.VMEM_SHARED((n_pad, d), jnp.float32),
            pltpu.VMEM_SHARED((n_pad,), jnp.float32),
        ],
    )
    def sc_kernel(src_hbm, dst_hbm, el_hbm, er_hbm, g_hbm, feat_hbm,
                  z2_hbm, z1_hbm, acc_out, s_out,
                  src_v, dst_v,
                  srcc_a, dstc_a, elg_a, erg_a, rows_a, sem_a,
                  srcc_b, dstc_b, elg_b, erg_b, rows_b, sem_b,
                  p_v, g_v, st1_v, acc_sp, s_sp):
        buf_a = (srcc_a, dstc_a, elg_a, erg_a, rows_a, sem_a)
        buf_b = (srcc_b, dstc_b, elg_b, erg_b, rows_b, sem_b)
        cid = lax.axis_index("c")
        tid = lax.axis_index("s")
        base = (cid * NS + tid) * epw

        pltpu.sync_copy(z2_hbm, rows_a)
        pltpu.sync_copy(z1_hbm, st1_v)
        for k in range(rpt // CH):
            pltpu.sync_copy(rows_a,
                            acc_sp.at[pl.ds(tid * rpt + k * CH, CH), :])
        pltpu.sync_copy(st1_v, s_sp.at[pl.ds(tid * rpt, rpt)])

        pltpu.sync_copy(src_hbm.at[pl.ds(base, epw)], src_v)
        pltpu.sync_copy(dst_hbm.at[pl.ds(base, epw)], dst_v)
        pltpu.sync_copy(g_hbm, g_v)
        gvec = g_v[...]

        def fire(c, buf):
            srcc, dstc, elg, erg, rows, sem = buf
            off = c * CH
            for v in range(CH // L):
                sl_in = pl.ds(off + v * L, L)
                sl_out = pl.ds(v * L, L)
                srcc[sl_out] = src_v[sl_in]
                dstc[sl_out] = dst_v[sl_in]
            pltpu.async_copy(el_hbm.at[srcc], elg, sem)
            pltpu.async_copy(er_hbm.at[dstc], erg, sem)
            pltpu.async_copy(feat_hbm.at[srcc], rows, sem)

        def process(buf):
            srcc, dstc, elg, erg, rows, sem = buf
            pltpu.make_async_copy(el_hbm.at[srcc], elg, sem).wait()
            pltpu.make_async_copy(er_hbm.at[dstc], erg, sem).wait()
            for v in range(CH // L):
                sl = pl.ds(v * L, L)
                e16 = elg[sl] + erg[sl]
                e16 = jnp.where(e16 > 0, e16, 0.01 * e16)
                p_v[sl] = jnp.exp(e16 - gvec)
            pltpu.make_async_copy(feat_hbm.at[srcc], rows, sem).wait()
            cpp = pltpu.async_copy(p_v, s_sp.at[dstc], sem, add=True)
            for vb in range(CH // L):
                p16 = p_v[pl.ds(vb * L, L)]
                for jj in range(L):
                    j = vb * L + jj
                    a = p16[jj]
                    for r in range(d // L):
                        sl = pl.ds(r * L, L)
                        rows[j, sl] = rows[j, sl] * a
            cpp.wait()
            pltpu.sync_copy(rows, acc_sp.at[dstc], add=True)

        plsc.subcore_barrier()

        fire(0, buf_a)

        def outer(c2, carry):
            c = 2 * c2
            fire(c + 1, buf_b)
            process(buf_a)
            fire(c + 2, buf_a)
            process(buf_b)
            return carry

        lax.fori_loop(0, (nch - 1) // 2, outer, 0)
        process(buf_a)

        plsc.subcore_barrier()

        o = cid * n_pad + tid * rpt
        pltpu.sync_copy(s_sp.at[pl.ds(tid * rpt, rpt)], st1_v)
        pltpu.sync_copy(st1_v, s_out.at[pl.ds(o, rpt)])
        for k in range(rpt // CH):
            pltpu.sync_copy(acc_sp.at[pl.ds(tid * rpt + k * CH, CH), :],
                            rows_a)
            pltpu.sync_copy(rows_a, acc_out.at[pl.ds(o + k * CH, CH), :])

    return sc_kernel


def kernel(edge_index, feat_src, feat_dst, attn_l, attn_r):
    n, d = feat_src.shape
    e = edge_index.shape[1]
    n_pad = ((n + NS * 8 - 1) // (NS * 8)) * (NS * 8)
    assert e % (NC * NS * CH) == 0

    src = edge_index[0]
    dst = edge_index[1]

    el, er, g = pl.pallas_call(
        _prep_body,
        out_shape=[
            jax.ShapeDtypeStruct((n,), jnp.float32),
            jax.ShapeDtypeStruct((n,), jnp.float32),
            jax.ShapeDtypeStruct((L,), jnp.float32),
        ],
    )(feat_src, feat_dst, attn_l, attn_r)

    rpt = n_pad // NS
    z2 = jnp.zeros((CH, d), jnp.float32)
    z1 = jnp.zeros((rpt,), jnp.float32)

    sc_kernel = _make_sc_kernel(n, e, d, n_pad)
    acc2, s2 = sc_kernel(src, dst, el, er, g, feat_src, z2, z1)

    out = pl.pallas_call(
        functools.partial(_finalize_body, n),
        out_shape=jax.ShapeDtypeStruct((n, d), jnp.float32),
    )(acc2.reshape(NC, n_pad, d), s2.reshape(NC, n_pad))
    return out
